# baseline (speedup 1.0000x reference)
import math

import jax
import jax.numpy as jnp
from jax import lax
from jax.experimental import pallas as pl
from jax.experimental.pallas import tpu as pltpu
from jax.experimental.pallas import tpu_sc as plsc

N = 2048
SIZE = 32000
SMOOTHING = 0.1
EPS = SMOOTHING / (SIZE - 1)
CONF = 1.0 - SMOOTHING
C0 = (SIZE - 1) * EPS * math.log(EPS) + CONF * math.log(CONF)

NC = 2
NS = 16
LANES = 16
NW = NC * NS
BPW = N // NW

RSC = 512
RPS = RSC // NW
UNROLL = 8
VECS = SIZE // (LANES * UNROLL)

ROWS_PER_BLOCK = 32
NSTREAMS = 4
GRID = (N - RSC) // (ROWS_PER_BLOCK * NSTREAMS)


def _sc_gather_body(xflat, tgt, out, tgt_v, idx_v, val_v, sem):
    wid = lax.axis_index("s") * NC + lax.axis_index("c")
    base = wid * BPW
    pltpu.sync_copy(tgt.at[pl.ds(base, BPW)], tgt_v)
    for j in range(BPW // LANES):
        rows = (base + j * LANES) + lax.broadcasted_iota(jnp.int32, (LANES,), 0)
        idx_v[pl.ds(j * LANES, LANES)] = rows * SIZE + tgt_v[pl.ds(j * LANES, LANES)]
    pltpu.async_copy(xflat.at[idx_v], val_v, sem).wait()
    pltpu.sync_copy(val_v, out.at[pl.ds(base, BPW)])


def _sc_gather(xflat, tgt):
    k = pl.kernel(
        _sc_gather_body,
        out_type=jax.ShapeDtypeStruct((N,), jnp.float32),
        mesh=plsc.VectorSubcoreMesh(core_axis_name="c", subcore_axis_name="s"),
        scratch_types=[
            pltpu.VMEM((BPW,), jnp.int32),
            pltpu.VMEM((BPW,), jnp.int32),
            pltpu.VMEM((BPW,), jnp.float32),
            pltpu.SemaphoreType.DMA,
        ],
    )
    return k(xflat, tgt)


def _sc_sum_body(xflat, out, buf0, buf1, stage, sem0, sem1):
    wid = lax.axis_index("s") * NC + lax.axis_index("c")
    row0 = wid * RPS
    bufs = (buf0, buf1)
    sems = (sem0, sem1)
    copies = [None, None]
    copies[0] = pltpu.async_copy(
        xflat.at[pl.ds(row0 * SIZE, SIZE)], bufs[0], sems[0]
    )
    accs = tuple(jnp.zeros((LANES,), jnp.float32) for _ in range(UNROLL))
    for r in range(RPS):
        cur = bufs[r % 2]
        if r + 1 < RPS:
            copies[(r + 1) % 2] = pltpu.async_copy(
                xflat.at[pl.ds((row0 + r + 1) * SIZE, SIZE)],
                bufs[(r + 1) % 2],
                sems[(r + 1) % 2],
            )
        copies[r % 2].wait()

        def body(i, accs_, cur=cur):
            base = i * (LANES * UNROLL)
            return tuple(
                a + cur[pl.ds(base + k * LANES, LANES)]
                for k, a in enumerate(accs_)
            )

        accs = lax.fori_loop(0, VECS, body, accs)
    total = accs[0]
    for a in accs[1:]:
        total = total + a
    stage[...] = total
    pltpu.sync_copy(stage, out.at[pl.ds(wid * LANES, LANES)])


def _sc_sum(xflat):
    k = pl.kernel(
        _sc_sum_body,
        out_type=jax.ShapeDtypeStruct((NW * LANES,), jnp.float32),
        mesh=plsc.VectorSubcoreMesh(core_axis_name="c", subcore_axis_name="s"),
        scratch_types=[
            pltpu.VMEM((SIZE,), jnp.float32),
            pltpu.VMEM((SIZE,), jnp.float32),
            pltpu.VMEM((LANES,), jnp.float32),
            pltpu.SemaphoreType.DMA,
            pltpu.SemaphoreType.DMA,
        ],
    )
    return k(xflat)


def _tc_loss_body(*refs):
    x_refs = refs[:NSTREAMS]
    g_ref = refs[NSTREAMS]
    out_ref = refs[NSTREAMS + 1]
    i = pl.program_id(0)

    @pl.when(i == 0)
    def _init():
        out_ref[0, 0] = jnp.float32(0.0)

    acc = jnp.float32(0.0)
    for r in x_refs:
        acc += jnp.sum(r[...])
    out_ref[0, 0] += acc

    @pl.when(i == GRID - 1)
    def _fin():
        s = out_ref[0, 0]
        g = jnp.sum(g_ref[...])
        out_ref[0, 0] = (
            jnp.float32(C0)
            - jnp.float32(EPS) * (s / N)
            + jnp.float32(EPS - CONF) * (g / N)
        )


def _tc_loss(x, gvals):
    g2 = gvals.reshape(LANES, N // LANES)
    base_blk = RSC // ROWS_PER_BLOCK
    x_specs = [
        pl.BlockSpec(
            (ROWS_PER_BLOCK, SIZE), lambda i, k=k: (base_blk + k * GRID + i, 0)
        )
        for k in range(NSTREAMS)
    ]
    out = pl.pallas_call(
        _tc_loss_body,
        grid=(GRID,),
        in_specs=x_specs + [pl.BlockSpec((LANES, N // LANES), lambda i: (0, 0))],
        out_specs=pl.BlockSpec(memory_space=pltpu.SMEM),
        out_shape=jax.ShapeDtypeStruct((1, 1), jnp.float32),
    )(*([x] * NSTREAMS), g2)
    return out[0, 0]


def kernel(x, target):
    tgt = target.astype(jnp.int32)
    xflat = x.reshape(N * SIZE)
    gp = _sc_gather(xflat, tgt)
    sp = _sc_sum(xflat)
    loss_tc = _tc_loss(x, gp)
    return loss_tc - jnp.float32(EPS) * (jnp.sum(sp) / N)


# speedup vs baseline: 1.0015x; 1.0015x over previous
import math

import jax
import jax.numpy as jnp
from jax import lax
from jax.experimental import pallas as pl
from jax.experimental.pallas import tpu as pltpu
from jax.experimental.pallas import tpu_sc as plsc

N = 2048
SIZE = 32000
SMOOTHING = 0.1
EPS = SMOOTHING / (SIZE - 1)
CONF = 1.0 - SMOOTHING
C0 = (SIZE - 1) * EPS * math.log(EPS) + CONF * math.log(CONF)

NC = 2
NS = 16
LANES = 16
NW = NC * NS
BPW = N // NW

RSC = 512
RPS = RSC // NW
UNROLL = 8
VECS = SIZE // (LANES * UNROLL)

ROWS_PER_BLOCK = 16
NSTREAMS = 8
GRID = (N - RSC) // (ROWS_PER_BLOCK * NSTREAMS)


def _sc_gather_body(xflat, tgt, out, tgt_v, idx_v, val_v, sem):
    wid = lax.axis_index("s") * NC + lax.axis_index("c")
    base = wid * BPW
    pltpu.sync_copy(tgt.at[pl.ds(base, BPW)], tgt_v)
    for j in range(BPW // LANES):
        rows = (base + j * LANES) + lax.broadcasted_iota(jnp.int32, (LANES,), 0)
        idx_v[pl.ds(j * LANES, LANES)] = rows * SIZE + tgt_v[pl.ds(j * LANES, LANES)]
    pltpu.async_copy(xflat.at[idx_v], val_v, sem).wait()
    pltpu.sync_copy(val_v, out.at[pl.ds(base, BPW)])


def _sc_gather(xflat, tgt):
    k = pl.kernel(
        _sc_gather_body,
        out_type=jax.ShapeDtypeStruct((N,), jnp.float32),
        mesh=plsc.VectorSubcoreMesh(core_axis_name="c", subcore_axis_name="s"),
        scratch_types=[
            pltpu.VMEM((BPW,), jnp.int32),
            pltpu.VMEM((BPW,), jnp.int32),
            pltpu.VMEM((BPW,), jnp.float32),
            pltpu.SemaphoreType.DMA,
        ],
    )
    return k(xflat, tgt)


def _sc_sum_body(xflat, out, buf0, buf1, stage, sem0, sem1):
    wid = lax.axis_index("s") * NC + lax.axis_index("c")
    row0 = wid * RPS
    bufs = (buf0, buf1)
    sems = (sem0, sem1)
    copies = [None, None]
    copies[0] = pltpu.async_copy(
        xflat.at[pl.ds(row0 * SIZE, SIZE)], bufs[0], sems[0]
    )
    accs = tuple(jnp.zeros((LANES,), jnp.float32) for _ in range(UNROLL))
    for r in range(RPS):
        cur = bufs[r % 2]
        if r + 1 < RPS:
            copies[(r + 1) % 2] = pltpu.async_copy(
                xflat.at[pl.ds((row0 + r + 1) * SIZE, SIZE)],
                bufs[(r + 1) % 2],
                sems[(r + 1) % 2],
            )
        copies[r % 2].wait()

        def body(i, accs_, cur=cur):
            base = i * (LANES * UNROLL)
            return tuple(
                a + cur[pl.ds(base + k * LANES, LANES)]
                for k, a in enumerate(accs_)
            )

        accs = lax.fori_loop(0, VECS, body, accs)
    total = accs[0]
    for a in accs[1:]:
        total = total + a
    stage[...] = total
    pltpu.sync_copy(stage, out.at[pl.ds(wid * LANES, LANES)])


def _sc_sum(xflat):
    k = pl.kernel(
        _sc_sum_body,
        out_type=jax.ShapeDtypeStruct((NW * LANES,), jnp.float32),
        mesh=plsc.VectorSubcoreMesh(core_axis_name="c", subcore_axis_name="s"),
        scratch_types=[
            pltpu.VMEM((SIZE,), jnp.float32),
            pltpu.VMEM((SIZE,), jnp.float32),
            pltpu.VMEM((LANES,), jnp.float32),
            pltpu.SemaphoreType.DMA,
            pltpu.SemaphoreType.DMA,
        ],
    )
    return k(xflat)


def _tc_loss_body(*refs):
    x_refs = refs[:NSTREAMS]
    g_ref = refs[NSTREAMS]
    out_ref = refs[NSTREAMS + 1]
    i = pl.program_id(0)

    @pl.when(i == 0)
    def _init():
        out_ref[0, 0] = jnp.float32(0.0)

    acc = jnp.float32(0.0)
    for r in x_refs:
        acc += jnp.sum(r[...])
    out_ref[0, 0] += acc

    @pl.when(i == GRID - 1)
    def _fin():
        s = out_ref[0, 0]
        g = jnp.sum(g_ref[...])
        out_ref[0, 0] = (
            jnp.float32(C0)
            - jnp.float32(EPS) * (s / N)
            + jnp.float32(EPS - CONF) * (g / N)
        )


def _tc_loss(x, gvals):
    g2 = gvals.reshape(LANES, N // LANES)
    base_blk = RSC // ROWS_PER_BLOCK
    x_specs = [
        pl.BlockSpec(
            (ROWS_PER_BLOCK, SIZE), lambda i, k=k: (base_blk + k * GRID + i, 0)
        )
        for k in range(NSTREAMS)
    ]
    out = pl.pallas_call(
        _tc_loss_body,
        grid=(GRID,),
        in_specs=x_specs + [pl.BlockSpec((LANES, N // LANES), lambda i: (0, 0))],
        out_specs=pl.BlockSpec(memory_space=pltpu.SMEM),
        out_shape=jax.ShapeDtypeStruct((1, 1), jnp.float32),
    )(*([x] * NSTREAMS), g2)
    return out[0, 0]


def kernel(x, target):
    tgt = target.astype(jnp.int32)
    xflat = x.reshape(N * SIZE)
    gp = _sc_gather(xflat, tgt)
    sp = _sc_sum(xflat)
    loss_tc = _tc_loss(x, gp)
    return loss_tc - jnp.float32(EPS) * (jnp.sum(sp) / N)
